# baseline (speedup 1.0000x reference)
import functools

import jax
import jax.numpy as jnp
from jax import lax
from jax.experimental import pallas as pl
from jax.experimental.pallas import tpu as pltpu

B = 16
C = 80
H = 128
W = 128
HW = H * W
CBLK = 8
NCB = C // CBLK
TOPK = 100
NMS_THRESH = 0.45


def _win5_w(a):
    cb, h, w = a.shape
    z1 = jnp.zeros((cb, h, 1), a.dtype)
    z2 = jnp.zeros((cb, h, 2), a.dtype)
    l1 = jnp.concatenate([a[:, :, 1:], z1], axis=2)
    l2 = jnp.concatenate([a[:, :, 2:], z2], axis=2)
    r1 = jnp.concatenate([z1, a[:, :, :-1]], axis=2)
    r2 = jnp.concatenate([z2, a[:, :, :-2]], axis=2)
    return jnp.maximum(jnp.maximum(jnp.maximum(l1, l2), jnp.maximum(r1, r2)), a)


def _win5_h(a):
    cb, h, w = a.shape
    z1 = jnp.zeros((cb, 1, w), a.dtype)
    z2 = jnp.zeros((cb, 2, w), a.dtype)
    u1 = jnp.concatenate([a[:, 1:, :], z1], axis=1)
    u2 = jnp.concatenate([a[:, 2:, :], z2], axis=1)
    d1 = jnp.concatenate([z1, a[:, :-1, :]], axis=1)
    d2 = jnp.concatenate([z2, a[:, :-2, :]], axis=1)
    return jnp.maximum(jnp.maximum(jnp.maximum(u1, u2), jnp.maximum(d1, d2)), a)


def _fused_body(cls_ref, m4_ref, code_ref, t_ref, s_scr):
    cb = pl.program_id(1)
    x = cls_ref[0]
    s = jax.nn.sigmoid(x)
    hm = _win5_h(_win5_w(s))
    masked = jnp.where(s == hm, s, 0.0)

    a2 = masked.reshape(CBLK, 64, 2, 128)
    p = jnp.max(a2, axis=2)
    i2r = lax.broadcasted_iota(jnp.int32, (CBLK, 64, 2, 128), 2)
    rowbit = jnp.min(jnp.where(a2 == p[:, :, None, :], i2r, 2), axis=2)
    q = p.reshape(CBLK, 64, 64, 2)
    m4 = jnp.max(q, axis=3)
    i2 = lax.broadcasted_iota(jnp.int32, (CBLK, 64, 64, 2), 3)
    colbit = jnp.min(jnp.where(q == m4[..., None], i2, 2), axis=3)
    rb = rowbit.reshape(CBLK, 64, 64, 2)
    rbwin = jnp.sum(rb * (i2 == colbit[..., None]).astype(jnp.int32), axis=3)
    code = rbwin * 2 + colbit

    m4_ref[0] = m4
    code_ref[0] = code
    s_scr[pl.ds(cb * CBLK, CBLK), :] = jnp.max(m4, axis=2)

    @pl.when(cb == NCB - 1)
    def _():
        sb = lax.bitcast_convert_type(s_scr[...], jnp.int32)

        def body(_, carry):
            lo, hi = carry
            mid = (lo + hi) // 2
            cnt = jnp.sum((sb >= mid).astype(jnp.int32))
            ge = cnt >= TOPK
            return jnp.where(ge, mid, lo), jnp.where(ge, hi, mid)

        lo, hi = lax.fori_loop(0, 31, body, (jnp.int32(0), jnp.int32(0x3F800000)))
        t = lax.bitcast_convert_type(lo, jnp.float32)
        t_ref[0, 0, :] = jnp.full((16,), t, jnp.float32)


def _decode_body(txty_ref, twth_ref, boxes_ref):
    gx = lax.broadcasted_iota(jnp.int32, (H, W), 1).astype(jnp.float32)
    gy = lax.broadcasted_iota(jnp.int32, (H, W), 0).astype(jnp.float32)
    xs = (jax.nn.sigmoid(txty_ref[0, 0]) + gx) * 4.0
    ys = (jax.nn.sigmoid(txty_ref[0, 1]) + gy) * 4.0
    ws = jnp.exp(twth_ref[0, 0]) * 4.0
    hs = jnp.exp(twth_ref[0, 1]) * 4.0
    boxes_ref[0, 0] = jnp.clip((xs - ws / 2.0) / 512.0, 0.0, 1.0)
    boxes_ref[0, 1] = jnp.clip((ys - hs / 2.0) / 512.0, 0.0, 1.0)
    boxes_ref[0, 2] = jnp.clip((xs + ws / 2.0) / 512.0, 0.0, 1.0)
    boxes_ref[0, 3] = jnp.clip((ys + hs / 2.0) / 512.0, 0.0, 1.0)


@functools.partial(jax.jit, static_argnames=("interpret",))
def _stage1(cls_preds, txty_preds, twth_preds, interpret=False):
    m4, code, t_rep = pl.pallas_call(
        _fused_body,
        grid=(B, NCB),
        in_specs=[pl.BlockSpec((1, CBLK, H, W), lambda b, cb: (b, cb, 0, 0))],
        out_specs=[
            pl.BlockSpec((1, CBLK, 64, 64), lambda b, cb: (b, cb, 0, 0)),
            pl.BlockSpec((1, CBLK, 64, 64), lambda b, cb: (b, cb, 0, 0)),
            pl.BlockSpec((1, 1, 16), lambda b, cb: (b, 0, 0)),
        ],
        out_shape=[
            jax.ShapeDtypeStruct((B, C, 64, 64), jnp.float32),
            jax.ShapeDtypeStruct((B, C, 64, 64), jnp.int32),
            jax.ShapeDtypeStruct((B, 1, 16), jnp.float32),
        ],
        scratch_shapes=[pltpu.VMEM((C, 64), jnp.float32)],
        interpret=interpret,
    )(cls_preds)

    boxes = pl.pallas_call(
        _decode_body,
        grid=(B,),
        in_specs=[
            pl.BlockSpec((1, 2, H, W), lambda b: (b, 0, 0, 0)),
            pl.BlockSpec((1, 2, H, W), lambda b: (b, 0, 0, 0)),
        ],
        out_specs=pl.BlockSpec((1, 4, H, W), lambda b: (b, 0, 0, 0)),
        out_shape=jax.ShapeDtypeStruct((B, 4, H, W), jnp.float32),
        interpret=interpret,
    )(txty_preds, twth_preds)
    return m4, code, t_rep, boxes


def _nms_keep_jnp(boxes, clses):
    x1, y1, x2, y2 = boxes[:, 0], boxes[:, 1], boxes[:, 2], boxes[:, 3]
    areas = (x2 - x1) * (y2 - y1)
    xx1 = jnp.maximum(x1[:, None], x1[None, :])
    yy1 = jnp.maximum(y1[:, None], y1[None, :])
    xx2 = jnp.minimum(x2[:, None], x2[None, :])
    yy2 = jnp.minimum(y2[:, None], y2[None, :])
    w = jnp.maximum(1e-28, xx2 - xx1)
    h = jnp.maximum(1e-28, yy2 - yy1)
    inter = w * h
    iou = inter / (areas[:, None] + areas[None, :] - inter)
    same = clses[:, None] == clses[None, :]
    idx = jnp.arange(TOPK)

    def body(i, keep):
        sup = (iou[i] > NMS_THRESH) & same[i] & (idx > i) & keep[i]
        return keep & (~sup)

    return lax.fori_loop(0, TOPK, body, jnp.ones((TOPK,), dtype=bool))


def kernel(cls_preds, txty_preds, twth_preds):
    m4, code, t_rep, boxes = _stage1(cls_preds, txty_preds, twth_preds)
    t = t_rep[:, 0, 0:1]
    m4f = m4.reshape(B, -1)
    sel = jnp.where(m4f >= t, m4f, -1.0)
    vals, pos = lax.top_k(sel, TOPK)
    cd = jnp.take_along_axis(code.reshape(B, -1), pos, axis=1)
    c = pos // 4096
    rj = pos % 4096
    r = rj // 64
    j = rj % 64
    hw = (2 * r + cd // 2) * W + 2 * j + cd % 2
    bsel = jnp.take_along_axis(boxes.reshape(B, 4, HW), hw[:, None, :], axis=2)
    bsel = jnp.transpose(bsel, (0, 2, 1))
    keep = jax.vmap(_nms_keep_jnp)(bsel, c)
    scores_out = vals * keep.astype(vals.dtype)
    boxes_out = bsel * keep[:, :, None].astype(bsel.dtype)
    return scores_out, boxes_out, c
